# Initial kernel scaffold; baseline (speedup 1.0000x reference)
import jax
import jax.numpy as jnp
from jax.experimental import pallas as pl


def kernel(inputs, epoch, table, W3, b3):
    raise NotImplementedError("write your pallas kernel here")


# speedup vs baseline: 2.1685x; 2.1685x over previous
import functools

import jax
import jax.numpy as jnp
from jax import lax
from jax.experimental import pallas as pl
from jax.experimental.pallas import tpu as pltpu
from jax.experimental.pallas import tpu_sc as plsc

NUM_CORES = 2
NUM_SUBCORES = 16
NUM_WORKERS = NUM_CORES * NUM_SUBCORES
LANES = 16

VOCAB = 1000000
EMB_DIM = 512
BATCH = 16384

PAIRS_PER_WORKER = BATCH // NUM_WORKERS
CHUNK_PAIRS = 32
CHUNK_ROWS = 2 * CHUNK_PAIRS
N_CHUNKS = PAIRS_PER_WORKER // CHUNK_PAIRS
NBUF = 2
PAIR_GROUP = 4
N_SLICES = EMB_DIM // LANES


def _sc_body(table, idxf, w3f, b3p, out,
             idx_v, w3_v, wa0_v, wa1_v, wb0_v, wb1_v,
             x0_v, x1_v, out_v, b3_v, buf0, buf1, sem0, sem1):
  wid = lax.axis_index("s") * NUM_CORES + lax.axis_index("c")
  flat_base = wid * (2 * PAIRS_PER_WORKER)

  pltpu.sync_copy(idxf.at[pl.ds(flat_base, 2 * PAIRS_PER_WORKER)], idx_v)
  pltpu.sync_copy(w3f, w3_v)
  pltpu.sync_copy(b3p, b3_v)

  def w_prep(j, _):
    d2 = 2 * (16 * j + lax.iota(jnp.int32, 16))
    wd0 = plsc.load_gather(w3_v, [d2])
    ws0 = plsc.load_gather(w3_v, [d2 + 1024])
    wa0 = plsc.load_gather(w3_v, [d2 + 2048])
    wb0 = plsc.load_gather(w3_v, [d2 + 3072])
    wd1 = plsc.load_gather(w3_v, [d2 + 1])
    ws1 = plsc.load_gather(w3_v, [d2 + 1025])
    wa1 = plsc.load_gather(w3_v, [d2 + 2049])
    wb1 = plsc.load_gather(w3_v, [d2 + 3073])
    sl = pl.ds(16 * j, 16)
    wa0_v[sl] = wd0 + ws0 + wa0
    wa1_v[sl] = wd1 + ws1 + wa1
    wb0_v[sl] = ws0 - wd0 + wb0
    wb1_v[sl] = ws1 - wd1 + wb1
    return 0

  lax.fori_loop(0, N_SLICES, w_prep, 0)

  bufs = (buf0, buf1)
  sems = (sem0, sem1)

  def gather_chunk(cc, b):
    idx_sl = idx_v.at[pl.ds(cc * CHUNK_ROWS, CHUNK_ROWS)]
    pltpu.make_async_copy(table.at[idx_sl], bufs[b], sems[b]).start()

  def wait_chunk(cc, b):
    idx_sl = idx_v.at[pl.ds(cc * CHUNK_ROWS, CHUNK_ROWS)]
    pltpu.make_async_copy(table.at[idx_sl], bufs[b], sems[b]).wait()

  for b in range(NBUF):
    gather_chunk(jnp.int32(b), b)

  zero = jnp.zeros((LANES,), jnp.float32)

  def compute_chunk(cc, buf):
    def group_body(g, _):
      i0 = g * PAIR_GROUP
      acc = [[zero, zero] for _ in range(PAIR_GROUP)]

      def a_side(j, accs):
        sl = pl.ds(16 * j, 16)
        w0 = wa0_v[sl]
        w1 = wa1_v[sl]
        out_accs = []
        for q in range(PAIR_GROUP):
          va = buf[(i0 + q) * 2, sl]
          out_accs.append((accs[2 * q] + va * w0, accs[2 * q + 1] + va * w1))
        return tuple(x for pair in out_accs for x in pair)

      def b_side(j, accs):
        sl = pl.ds(16 * j, 16)
        w0 = wb0_v[sl]
        w1 = wb1_v[sl]
        out_accs = []
        for q in range(PAIR_GROUP):
          vb = buf[(i0 + q) * 2 + 1, sl]
          out_accs.append((accs[2 * q] + vb * w0, accs[2 * q + 1] + vb * w1))
        return tuple(x for pair in out_accs for x in pair)

      flat = tuple(x for pair in acc for x in pair)
      flat = lax.fori_loop(0, N_SLICES, a_side, flat)
      flat = lax.fori_loop(0, N_SLICES, b_side, flat)
      last = lax.iota(jnp.int32, 16) == 15
      for q in range(PAIR_GROUP):
        p = jnp.broadcast_to(cc * CHUNK_PAIRS + i0 + q, (16,))
        plsc.store_scatter(x0_v, [p], plsc.cumsum(flat[2 * q]), mask=last)
        plsc.store_scatter(x1_v, [p], plsc.cumsum(flat[2 * q + 1]), mask=last)
      return 0

    lax.fori_loop(0, CHUNK_PAIRS // PAIR_GROUP, group_body, 0)

  def ring_body(g, _):
    for b in range(NBUF):
      cc = g * NBUF + b
      wait_chunk(cc, b)
      compute_chunk(cc, bufs[b])

      @pl.when(cc + NBUF < N_CHUNKS)
      def _():
        gather_chunk(cc + NBUF, b)

    return 0

  lax.fori_loop(0, N_CHUNKS // NBUF, ring_body, 0)

  bv = b3_v[pl.ds(0, 16)]
  b0 = bv[0]
  b1 = bv[1]

  def epilogue(j, _):
    sl = pl.ds(16 * j, 16)
    x0 = x0_v[sl] + b0
    x1 = x1_v[sl] + b1
    m = jnp.maximum(x0, x1)
    t = jnp.exp(-jnp.abs(x0 - x1))
    z = t / (2.0 + t)
    z2 = z * z
    log1p_t = 2.0 * z * (1.0 + z2 * (1.0 / 3.0 + z2 * (0.2 + z2 * (1.0 / 7.0 + z2 * (1.0 / 9.0)))))
    lse = m + log1p_t
    ids = 2 * (16 * j + lax.iota(jnp.int32, 16))
    plsc.store_scatter(out_v, [ids], x0 - lse)
    plsc.store_scatter(out_v, [ids + 1], x1 - lse)
    return 0

  lax.fori_loop(0, PAIRS_PER_WORKER // 16, epilogue, 0)

  pltpu.sync_copy(out_v, out.at[pl.ds(wid * (2 * PAIRS_PER_WORKER),
                                      2 * PAIRS_PER_WORKER)])


@functools.partial(
    pl.kernel,
    out_type=jax.ShapeDtypeStruct((2 * BATCH,), jnp.float32),
    mesh=plsc.VectorSubcoreMesh(core_axis_name="c", subcore_axis_name="s"),
    scratch_types=[
        pltpu.VMEM((2 * PAIRS_PER_WORKER,), jnp.int32),
        pltpu.VMEM((4096,), jnp.float32),
        pltpu.VMEM((EMB_DIM,), jnp.float32),
        pltpu.VMEM((EMB_DIM,), jnp.float32),
        pltpu.VMEM((EMB_DIM,), jnp.float32),
        pltpu.VMEM((EMB_DIM,), jnp.float32),
        pltpu.VMEM((PAIRS_PER_WORKER,), jnp.float32),
        pltpu.VMEM((PAIRS_PER_WORKER,), jnp.float32),
        pltpu.VMEM((2 * PAIRS_PER_WORKER,), jnp.float32),
        pltpu.VMEM((LANES,), jnp.float32),
        pltpu.VMEM((CHUNK_ROWS, EMB_DIM), jnp.float32),
        pltpu.VMEM((CHUNK_ROWS, EMB_DIM), jnp.float32),
        pltpu.SemaphoreType.DMA,
        pltpu.SemaphoreType.DMA,
    ],
    compiler_params=pltpu.CompilerParams(needs_layout_passes=False),
)
def _siamese_sc(table, idxf, w3f, b3p, out, *scratch):
  _sc_body(table, idxf, w3f, b3p, out, *scratch)


def kernel(inputs, epoch, table, W3, b3):
  del epoch
  idx_flat = inputs.reshape(-1).astype(jnp.int32)
  w3_flat = W3.reshape(-1)
  b3_pad = jnp.pad(b3, (0, LANES - b3.shape[0]))
  out_flat = _siamese_sc(table, idx_flat, w3_flat, b3_pad)
  return out_flat.reshape(BATCH, 2)
